# baseline (speedup 1.0000x reference)
import functools

import jax
import jax.numpy as jnp
from jax import lax
from jax.experimental import pallas as pl
from jax.experimental.pallas import tpu as pltpu
from jax.experimental.pallas import tpu_sc as plsc

DIM = 128
BATCH = 16384
TABLE_ROWS = 1000
NUM_CORES = 2
NUM_SUBCORES = 16
NUM_WORKERS = NUM_CORES * NUM_SUBCORES
B_PER_W = BATCH // NUM_WORKERS
CHUNK = 64
N_CHUNKS = B_PER_W // CHUNK
STAGE_ROWS = 128
STAGE_TILES_FULL = TABLE_ROWS // STAGE_ROWS
STAGE_REM = TABLE_ROWS - STAGE_TILES_FULL * STAGE_ROWS


@jax.jit
def _pe_lookup(pe_matrix, timestep):
    mesh = plsc.VectorSubcoreMesh(core_axis_name="c", subcore_axis_name="s")

    @functools.partial(
        pl.kernel,
        mesh=mesh,
        out_type=jax.ShapeDtypeStruct((BATCH, DIM), jnp.float32),
        scratch_types=[
            pltpu.VMEM_SHARED((TABLE_ROWS, DIM), jnp.float32),
            pltpu.VMEM((B_PER_W,), jnp.int32),
            pltpu.VMEM((B_PER_W, DIM), jnp.float32),
        ] + [pltpu.SemaphoreType.DMA] * (N_CHUNKS + 1),
    )
    def k(table_hbm, idx_hbm, out_hbm, table_s, idx_v, rows_v, *sems):
        gsems, wsem = sems[:N_CHUNKS], sems[N_CHUNKS]
        cid = lax.axis_index("c")
        sid = lax.axis_index("s")
        wid = sid * NUM_CORES + cid
        base = wid * B_PER_W

        pltpu.sync_copy(idx_hbm.at[pl.ds(base, B_PER_W)], idx_v)

        gathers = []
        for j in range(N_CHUNKS):
            gathers.append(pltpu.async_copy(
                table_hbm.at[idx_v.at[pl.ds(j * CHUNK, CHUNK)]],
                rows_v.at[pl.ds(j * CHUNK, CHUNK)],
                gsems[j],
            ))
        writes = []
        for j in range(N_CHUNKS):
            gathers[j].wait()
            writes.append(pltpu.async_copy(
                rows_v.at[pl.ds(j * CHUNK, CHUNK)],
                out_hbm.at[pl.ds(base + j * CHUNK, CHUNK)],
                wsem,
            ))
        for w in writes:
            w.wait()

    return k(pe_matrix, timestep)


def kernel(pe_matrix, timestep):
    return _pe_lookup(pe_matrix, timestep.astype(jnp.int32))


# speedup vs baseline: 1.2728x; 1.2728x over previous
import functools

import jax
import jax.numpy as jnp
from jax import lax
from jax.experimental import pallas as pl
from jax.experimental.pallas import tpu as pltpu
from jax.experimental.pallas import tpu_sc as plsc

DIM = 128
BATCH = 16384
TABLE_ROWS = 1000
NUM_CORES = 2
NUM_SUBCORES = 16
NUM_WORKERS = NUM_CORES * NUM_SUBCORES
B_PER_W = BATCH // NUM_WORKERS
CHUNK = 64
N_CHUNKS = B_PER_W // CHUNK
STAGE_ROWS = 64
STAGE_TILES_FULL = TABLE_ROWS // STAGE_ROWS
STAGE_REM = TABLE_ROWS - STAGE_TILES_FULL * STAGE_ROWS


@jax.jit
def _pe_lookup(pe_matrix, timestep):
    mesh = plsc.VectorSubcoreMesh(core_axis_name="c", subcore_axis_name="s")

    @functools.partial(
        pl.kernel,
        mesh=mesh,
        out_type=jax.ShapeDtypeStruct((BATCH, DIM), jnp.float32),
        scratch_types=[
            pltpu.VMEM_SHARED((TABLE_ROWS, DIM), jnp.float32),
            pltpu.VMEM((B_PER_W,), jnp.int32),
            pltpu.VMEM((B_PER_W, DIM), jnp.float32),
        ] + [pltpu.SemaphoreType.DMA] * (N_CHUNKS + 2),
    )
    def k(table_hbm, idx_hbm, out_hbm, table_s, idx_v, rows_v, *sems):
        gsems, wsem, isem = sems[:N_CHUNKS], sems[N_CHUNKS], sems[N_CHUNKS + 1]
        cid = lax.axis_index("c")
        sid = lax.axis_index("s")
        wid = sid * NUM_CORES + cid
        base = wid * B_PER_W

        idx_copy = pltpu.async_copy(idx_hbm.at[pl.ds(base, B_PER_W)], idx_v,
                                    isem)

        @pl.when(sid < STAGE_TILES_FULL)
        def _stage():
            r0 = pl.multiple_of(sid * STAGE_ROWS, STAGE_ROWS)
            pltpu.sync_copy(table_hbm.at[pl.ds(r0, STAGE_ROWS)],
                            table_s.at[pl.ds(r0, STAGE_ROWS)])

        @pl.when(sid == STAGE_TILES_FULL)
        def _stage_rem():
            r0 = STAGE_TILES_FULL * STAGE_ROWS
            pltpu.sync_copy(table_hbm.at[pl.ds(r0, STAGE_REM)],
                            table_s.at[pl.ds(r0, STAGE_REM)])

        idx_copy.wait()
        plsc.subcore_barrier()

        gathers = []
        for j in range(N_CHUNKS):
            gathers.append(pltpu.async_copy(
                table_s.at[idx_v.at[pl.ds(j * CHUNK, CHUNK)]],
                rows_v.at[pl.ds(j * CHUNK, CHUNK)],
                gsems[j],
            ))
        writes = []
        for j in range(N_CHUNKS):
            gathers[j].wait()
            writes.append(pltpu.async_copy(
                rows_v.at[pl.ds(j * CHUNK, CHUNK)],
                out_hbm.at[pl.ds(base + j * CHUNK, CHUNK)],
                wsem,
            ))
        for w in writes:
            w.wait()

    return k(pe_matrix, timestep)


def kernel(pe_matrix, timestep):
    return _pe_lookup(pe_matrix, timestep.astype(jnp.int32))
